# Initial kernel scaffold; baseline (speedup 1.0000x reference)
import jax
import jax.numpy as jnp
from jax.experimental import pallas as pl


def kernel(x, edge_index, W1, b1, W2, b2):
    raise NotImplementedError("write your pallas kernel here")


# speedup vs baseline: 6.2115x; 6.2115x over previous
import functools

import jax
import jax.numpy as jnp
from jax import lax
from jax.experimental import pallas as pl
from jax.experimental.pallas import tpu as pltpu
from jax.experimental.pallas import tpu_sc as plsc

N_NODES = 10000
D = 128

NPAD = 10240
NBLK = NPAD // 128
NW = 32
B = 128
NCH = 80
EW = NCH * B
EPAD = NW * EW
ROWS_PER_TILE = NPAD // 16

_MESH = dict(core_axis_name="c", subcore_axis_name="s")


def _sc_degree(dst_flat):
  @functools.partial(
      pl.kernel,
      mesh=plsc.VectorSubcoreMesh(**_MESH),
      out_type=jax.ShapeDtypeStruct((2 * NPAD, D), jnp.float32),
      scratch_types=[
          pltpu.VMEM((B,), jnp.int32),
          pltpu.VMEM((B, D), jnp.float32),
          pltpu.VMEM_SHARED((NPAD, D), jnp.float32),
      ],
  )
  def k(dst_hbm, out_hbm, cidx, ones_v, acc):
    c = lax.axis_index("c")
    s = lax.axis_index("s")
    w = c * 16 + s

    def fill_zero(t, _):
      i = t // 8
      j = lax.rem(t, 8) * 16
      ones_v[i, pl.ds(j, 16)] = jnp.zeros((16,), jnp.float32)
      return 0
    lax.fori_loop(0, B * 8, fill_zero, 0)
    for t in range(ROWS_PER_TILE // B):
      pltpu.sync_copy(ones_v, acc.at[pl.ds(s * ROWS_PER_TILE + t * B, B)])
    plsc.subcore_barrier()

    def fill_ones(t, _):
      i = t // 8
      j = lax.rem(t, 8) * 16
      ones_v[i, pl.ds(j, 16)] = jnp.full((16,), 1.0, jnp.float32)
      return 0
    lax.fori_loop(0, B * 8, fill_ones, 0)

    base = w * EW

    def chunk(j, _):
      pltpu.sync_copy(dst_hbm.at[pl.ds(base + j * B, B)], cidx)
      pltpu.sync_copy(ones_v, acc.at[cidx], add=True)
      return 0
    lax.fori_loop(0, NCH, chunk, 0)

    plsc.subcore_barrier()
    for t in range(ROWS_PER_TILE // B):
      r = s * ROWS_PER_TILE + t * B
      pltpu.sync_copy(acc.at[pl.ds(r, B)], ones_v)
      pltpu.sync_copy(ones_v, out_hbm.at[pl.ds(c * NPAD + r, B)])

  return k(dst_flat)


def _sc_aggregate(hs, src_flat, dst_flat):
  @functools.partial(
      pl.kernel,
      mesh=plsc.VectorSubcoreMesh(**_MESH),
      out_type=jax.ShapeDtypeStruct((2 * NPAD, D), jnp.float32),
      scratch_types=[
          pltpu.VMEM((B,), jnp.int32),
          pltpu.VMEM((B,), jnp.int32),
          pltpu.VMEM((B, D), jnp.float32),
          pltpu.VMEM_SHARED((NPAD, D), jnp.float32),
          pltpu.SemaphoreType.DMA,
      ],
  )
  def k(hs_hbm, src_hbm, dst_hbm, out_hbm, sidx, didx, buf, acc, sem):
    c = lax.axis_index("c")
    s = lax.axis_index("s")
    w = c * 16 + s

    def fill_zero(t, _):
      i = t // 8
      j = lax.rem(t, 8) * 16
      buf[i, pl.ds(j, 16)] = jnp.zeros((16,), jnp.float32)
      return 0
    lax.fori_loop(0, B * 8, fill_zero, 0)

    for t in range(ROWS_PER_TILE // B):
      pltpu.sync_copy(buf, acc.at[pl.ds(s * ROWS_PER_TILE + t * B, B)])
    plsc.subcore_barrier()

    base = w * EW

    def chunk(j, _):
      pltpu.sync_copy(src_hbm.at[pl.ds(base + j * B, B)], sidx)
      pltpu.sync_copy(dst_hbm.at[pl.ds(base + j * B, B)], didx)
      pltpu.async_copy(hs_hbm.at[sidx], buf, sem).wait()
      pltpu.sync_copy(buf, acc.at[didx], add=True)
      return 0
    lax.fori_loop(0, NCH, chunk, 0)

    plsc.subcore_barrier()
    for t in range(ROWS_PER_TILE // B):
      r = s * ROWS_PER_TILE + t * B
      pltpu.sync_copy(acc.at[pl.ds(r, B)], buf)
      pltpu.sync_copy(buf, out_hbm.at[pl.ds(c * NPAD + r, B)])

  return k(hs, src_flat, dst_flat)


def _tc_prescale(x_pad, degp, w1):
  def body(x_ref, dp_ref, w_ref, hs_ref, dis_ref):
    deg = dp_ref[0, :, 0] + dp_ref[1, :, 0] + 1.0
    dis = lax.rsqrt(deg)
    h = jnp.dot(x_ref[...], w_ref[...], preferred_element_type=jnp.float32)
    hs_ref[...] = h * dis[:, None]
    dis_ref[...] = dis[:, None]

  return pl.pallas_call(
      body,
      grid=(NBLK,),
      in_specs=[
          pl.BlockSpec((128, D), lambda i: (i, 0)),
          pl.BlockSpec((2, 128, D), lambda i: (0, i, 0)),
          pl.BlockSpec((D, D), lambda i: (0, 0)),
      ],
      out_specs=[
          pl.BlockSpec((128, D), lambda i: (i, 0)),
          pl.BlockSpec((128, 1), lambda i: (i, 0)),
      ],
      out_shape=[
          jax.ShapeDtypeStruct((NPAD, D), jnp.float32),
          jax.ShapeDtypeStruct((NPAD, 1), jnp.float32),
      ],
  )(x_pad, degp, w1)


def _tc_mid(parts, hs1, dis, b1, w2):
  def body(p_ref, hs_ref, dis_ref, b_ref, w_ref, out_ref):
    agg = p_ref[0] + p_ref[1] + hs_ref[...]
    h = jnp.maximum(agg * dis_ref[...] + b_ref[...], 0.0)
    out_ref[...] = jnp.dot(
        h, w_ref[...], preferred_element_type=jnp.float32) * dis_ref[...]

  return pl.pallas_call(
      body,
      grid=(NBLK,),
      in_specs=[
          pl.BlockSpec((2, 128, D), lambda i: (0, i, 0)),
          pl.BlockSpec((128, D), lambda i: (i, 0)),
          pl.BlockSpec((128, 1), lambda i: (i, 0)),
          pl.BlockSpec((1, D), lambda i: (0, 0)),
          pl.BlockSpec((D, D), lambda i: (0, 0)),
      ],
      out_specs=pl.BlockSpec((128, D), lambda i: (i, 0)),
      out_shape=jax.ShapeDtypeStruct((NPAD, D), jnp.float32),
  )(parts, hs1, dis, b1, w2)


def _tc_final(parts, hs2, dis, b2):
  def body(p_ref, hs_ref, dis_ref, b_ref, out_ref):
    agg = p_ref[0] + p_ref[1] + hs_ref[...]
    out_ref[...] = agg * dis_ref[...] + b_ref[...]

  return pl.pallas_call(
      body,
      grid=(NBLK,),
      in_specs=[
          pl.BlockSpec((2, 128, D), lambda i: (0, i, 0)),
          pl.BlockSpec((128, D), lambda i: (i, 0)),
          pl.BlockSpec((128, 1), lambda i: (i, 0)),
          pl.BlockSpec((1, D), lambda i: (0, 0)),
      ],
      out_specs=pl.BlockSpec((128, D), lambda i: (i, 0)),
      out_shape=jax.ShapeDtypeStruct((NPAD, D), jnp.float32),
  )(parts, hs2, dis, b2)


def kernel(x, edge_index, W1, b1, W2, b2):
  src = edge_index[0].astype(jnp.int32)
  dst = edge_index[1].astype(jnp.int32)
  npad_e = EPAD - src.shape[0]
  src_flat = jnp.concatenate([src, jnp.zeros((npad_e,), jnp.int32)])
  dst_flat = jnp.concatenate([dst, jnp.full((npad_e,), N_NODES, jnp.int32)])
  x_pad = jnp.pad(x, ((0, NPAD - N_NODES), (0, 0)))
  b1r = b1.reshape(1, D)
  b2r = b2.reshape(1, D)

  degp = _sc_degree(dst_flat).reshape(2, NPAD, D)
  hs1, dis = _tc_prescale(x_pad, degp, W1)
  parts1 = _sc_aggregate(hs1, src_flat, dst_flat).reshape(2, NPAD, D)
  hs2 = _tc_mid(parts1, hs1, dis, b1r, W2)
  parts2 = _sc_aggregate(hs2, src_flat, dst_flat).reshape(2, NPAD, D)
  out = _tc_final(parts2, hs2, dis, b2r)
  return out[:N_NODES]


# speedup vs baseline: 7.6096x; 1.2251x over previous
import functools

import jax
import jax.numpy as jnp
from jax import lax
from jax.experimental import pallas as pl
from jax.experimental.pallas import tpu as pltpu
from jax.experimental.pallas import tpu_sc as plsc

N_NODES = 10000
D = 128

NPAD = 10240
NBLK = NPAD // 128
NW = 32
B = 128
NCH = 80
EW = NCH * B
EPAD = NW * EW
ROWS_PER_TILE = NPAD // 16

_MESH = dict(core_axis_name="c", subcore_axis_name="s")


def _sc_degree(dst_flat):
  @functools.partial(
      pl.kernel,
      mesh=plsc.VectorSubcoreMesh(**_MESH),
      out_type=jax.ShapeDtypeStruct((2 * NPAD, D), jnp.float32),
      scratch_types=[
          pltpu.VMEM((B,), jnp.int32),
          pltpu.VMEM((B,), jnp.int32),
          pltpu.VMEM((B, D), jnp.float32),
          pltpu.VMEM_SHARED((NPAD, D), jnp.float32),
          pltpu.SemaphoreType.DMA,
          pltpu.SemaphoreType.DMA,
      ],
  )
  def k(dst_hbm, out_hbm, cidx0, cidx1, ones_v, acc, t0, t1):
    c = lax.axis_index("c")
    s = lax.axis_index("s")
    w = c * 16 + s

    def fill_zero(t, _):
      i = t // 8
      j = lax.rem(t, 8) * 16
      ones_v[i, pl.ds(j, 16)] = jnp.zeros((16,), jnp.float32)
      return 0
    lax.fori_loop(0, B * 8, fill_zero, 0)
    for t in range(ROWS_PER_TILE // B):
      pltpu.sync_copy(ones_v, acc.at[pl.ds(s * ROWS_PER_TILE + t * B, B)])
    plsc.subcore_barrier()

    def fill_ones(t, _):
      i = t // 8
      j = lax.rem(t, 8) * 16
      ones_v[i, pl.ds(j, 16)] = jnp.full((16,), 1.0, jnp.float32)
      return 0
    lax.fori_loop(0, B * 8, fill_ones, 0)

    base = w * EW

    pltpu.sync_copy(dst_hbm.at[pl.ds(base, B)], cidx0)
    pltpu.async_copy(ones_v, acc.at[cidx0], t0, add=True)
    pltpu.sync_copy(dst_hbm.at[pl.ds(base + B, B)], cidx1)
    pltpu.async_copy(ones_v, acc.at[cidx1], t1, add=True)

    def body(g, _):
      j0 = 2 * g
      pltpu.make_async_copy(ones_v, acc.at[cidx0], t0).wait()
      pltpu.sync_copy(dst_hbm.at[pl.ds(base + (j0 + 2) * B, B)], cidx0)
      pltpu.async_copy(ones_v, acc.at[cidx0], t0, add=True)
      pltpu.make_async_copy(ones_v, acc.at[cidx1], t1).wait()
      pltpu.sync_copy(dst_hbm.at[pl.ds(base + (j0 + 3) * B, B)], cidx1)
      pltpu.async_copy(ones_v, acc.at[cidx1], t1, add=True)
      return 0
    lax.fori_loop(0, NCH // 2 - 1, body, 0)

    pltpu.make_async_copy(ones_v, acc.at[cidx0], t0).wait()
    pltpu.make_async_copy(ones_v, acc.at[cidx1], t1).wait()

    plsc.subcore_barrier()
    for t in range(ROWS_PER_TILE // B):
      r = s * ROWS_PER_TILE + t * B
      pltpu.sync_copy(acc.at[pl.ds(r, B)], ones_v)
      pltpu.sync_copy(ones_v, out_hbm.at[pl.ds(c * NPAD + r, B)])

  return k(dst_flat)


def _sc_aggregate(hs, src_flat, dst_flat):
  @functools.partial(
      pl.kernel,
      mesh=plsc.VectorSubcoreMesh(**_MESH),
      out_type=jax.ShapeDtypeStruct((2 * NPAD, D), jnp.float32),
      scratch_types=[
          pltpu.VMEM((B,), jnp.int32),
          pltpu.VMEM((B,), jnp.int32),
          pltpu.VMEM((B,), jnp.int32),
          pltpu.VMEM((B,), jnp.int32),
          pltpu.VMEM((B, D), jnp.float32),
          pltpu.VMEM((B, D), jnp.float32),
          pltpu.VMEM_SHARED((NPAD, D), jnp.float32),
          pltpu.SemaphoreType.DMA,
          pltpu.SemaphoreType.DMA,
      ],
  )
  def k(hs_hbm, src_hbm, dst_hbm, out_hbm,
        sidx0, sidx1, didx0, didx1, buf0, buf1, acc, g0, g1):
    c = lax.axis_index("c")
    s = lax.axis_index("s")
    w = c * 16 + s

    def fill_zero(t, _):
      i = t // 8
      j = lax.rem(t, 8) * 16
      buf0[i, pl.ds(j, 16)] = jnp.zeros((16,), jnp.float32)
      return 0
    lax.fori_loop(0, B * 8, fill_zero, 0)

    for t in range(ROWS_PER_TILE // B):
      pltpu.sync_copy(buf0, acc.at[pl.ds(s * ROWS_PER_TILE + t * B, B)])
    plsc.subcore_barrier()

    base = w * EW

    pltpu.sync_copy(src_hbm.at[pl.ds(base, B)], sidx0)
    pltpu.sync_copy(dst_hbm.at[pl.ds(base, B)], didx0)
    pltpu.async_copy(hs_hbm.at[sidx0], buf0, g0)
    pltpu.sync_copy(src_hbm.at[pl.ds(base + B, B)], sidx1)
    pltpu.sync_copy(dst_hbm.at[pl.ds(base + B, B)], didx1)
    pltpu.async_copy(hs_hbm.at[sidx1], buf1, g1)

    def body(g, _):
      j0 = 2 * g
      pltpu.make_async_copy(hs_hbm.at[sidx0], buf0, g0).wait()
      pltpu.sync_copy(buf0, acc.at[didx0], add=True)
      pltpu.sync_copy(src_hbm.at[pl.ds(base + (j0 + 2) * B, B)], sidx0)
      pltpu.sync_copy(dst_hbm.at[pl.ds(base + (j0 + 2) * B, B)], didx0)
      pltpu.async_copy(hs_hbm.at[sidx0], buf0, g0)

      pltpu.make_async_copy(hs_hbm.at[sidx1], buf1, g1).wait()
      pltpu.sync_copy(buf1, acc.at[didx1], add=True)
      pltpu.sync_copy(src_hbm.at[pl.ds(base + (j0 + 3) * B, B)], sidx1)
      pltpu.sync_copy(dst_hbm.at[pl.ds(base + (j0 + 3) * B, B)], didx1)
      pltpu.async_copy(hs_hbm.at[sidx1], buf1, g1)
      return 0
    lax.fori_loop(0, NCH // 2 - 1, body, 0)

    pltpu.make_async_copy(hs_hbm.at[sidx0], buf0, g0).wait()
    pltpu.sync_copy(buf0, acc.at[didx0], add=True)
    pltpu.make_async_copy(hs_hbm.at[sidx1], buf1, g1).wait()
    pltpu.sync_copy(buf1, acc.at[didx1], add=True)

    plsc.subcore_barrier()
    for t in range(ROWS_PER_TILE // B):
      r = s * ROWS_PER_TILE + t * B
      pltpu.sync_copy(acc.at[pl.ds(r, B)], buf0)
      pltpu.sync_copy(buf0, out_hbm.at[pl.ds(c * NPAD + r, B)])

  return k(hs, src_flat, dst_flat)


def _tc_prescale(x_pad, degp, w1):
  def body(x_ref, dp_ref, w_ref, hs_ref, dis_ref):
    deg = dp_ref[0, :, 0] + dp_ref[1, :, 0] + 1.0
    dis = lax.rsqrt(deg)
    h = jnp.dot(x_ref[...], w_ref[...], preferred_element_type=jnp.float32)
    hs_ref[...] = h * dis[:, None]
    dis_ref[...] = dis[:, None]

  return pl.pallas_call(
      body,
      grid=(NBLK,),
      in_specs=[
          pl.BlockSpec((128, D), lambda i: (i, 0)),
          pl.BlockSpec((2, 128, D), lambda i: (0, i, 0)),
          pl.BlockSpec((D, D), lambda i: (0, 0)),
      ],
      out_specs=[
          pl.BlockSpec((128, D), lambda i: (i, 0)),
          pl.BlockSpec((128, 1), lambda i: (i, 0)),
      ],
      out_shape=[
          jax.ShapeDtypeStruct((NPAD, D), jnp.float32),
          jax.ShapeDtypeStruct((NPAD, 1), jnp.float32),
      ],
  )(x_pad, degp, w1)


def _tc_mid(parts, hs1, dis, b1, w2):
  def body(p_ref, hs_ref, dis_ref, b_ref, w_ref, out_ref):
    agg = p_ref[0] + p_ref[1] + hs_ref[...]
    h = jnp.maximum(agg * dis_ref[...] + b_ref[...], 0.0)
    out_ref[...] = jnp.dot(
        h, w_ref[...], preferred_element_type=jnp.float32) * dis_ref[...]

  return pl.pallas_call(
      body,
      grid=(NBLK,),
      in_specs=[
          pl.BlockSpec((2, 128, D), lambda i: (0, i, 0)),
          pl.BlockSpec((128, D), lambda i: (i, 0)),
          pl.BlockSpec((128, 1), lambda i: (i, 0)),
          pl.BlockSpec((1, D), lambda i: (0, 0)),
          pl.BlockSpec((D, D), lambda i: (0, 0)),
      ],
      out_specs=pl.BlockSpec((128, D), lambda i: (i, 0)),
      out_shape=jax.ShapeDtypeStruct((NPAD, D), jnp.float32),
  )(parts, hs1, dis, b1, w2)


def _tc_final(parts, hs2, dis, b2):
  def body(p_ref, hs_ref, dis_ref, b_ref, out_ref):
    agg = p_ref[0] + p_ref[1] + hs_ref[...]
    out_ref[...] = agg * dis_ref[...] + b_ref[...]

  return pl.pallas_call(
      body,
      grid=(NBLK,),
      in_specs=[
          pl.BlockSpec((2, 128, D), lambda i: (0, i, 0)),
          pl.BlockSpec((128, D), lambda i: (i, 0)),
          pl.BlockSpec((128, 1), lambda i: (i, 0)),
          pl.BlockSpec((1, D), lambda i: (0, 0)),
      ],
      out_specs=pl.BlockSpec((128, D), lambda i: (i, 0)),
      out_shape=jax.ShapeDtypeStruct((NPAD, D), jnp.float32),
  )(parts, hs2, dis, b2)


def kernel(x, edge_index, W1, b1, W2, b2):
  src = edge_index[0].astype(jnp.int32)
  dst = edge_index[1].astype(jnp.int32)
  npad_e = EPAD - src.shape[0]
  src_flat = jnp.concatenate([src, jnp.zeros((npad_e,), jnp.int32)])
  dst_flat = jnp.concatenate([dst, jnp.full((npad_e,), N_NODES, jnp.int32)])
  x_pad = jnp.pad(x, ((0, NPAD - N_NODES), (0, 0)))
  b1r = b1.reshape(1, D)
  b2r = b2.reshape(1, D)

  degp = _sc_degree(dst_flat).reshape(2, NPAD, D)
  hs1, dis = _tc_prescale(x_pad, degp, W1)
  parts1 = _sc_aggregate(hs1, src_flat, dst_flat).reshape(2, NPAD, D)
  hs2 = _tc_mid(parts1, hs1, dis, b1r, W2)
  parts2 = _sc_aggregate(hs2, src_flat, dst_flat).reshape(2, NPAD, D)
  out = _tc_final(parts2, hs2, dis, b2r)
  return out[:N_NODES]
